# baseline (speedup 1.0000x reference)
import functools

import jax
import jax.numpy as jnp
from jax import lax
from jax.experimental import pallas as pl
from jax.experimental.pallas import tpu as pltpu
from jax.experimental.pallas import tpu_sc as plsc

BATCH = 16384
FACTOR = 32
VOCAB = 1000000
NUM_CORES = 2
NUM_SUBCORES = 16
LANES = 16
NUM_WORKERS = NUM_CORES * NUM_SUBCORES
B_PER_W = BATCH // NUM_WORKERS
GROUPS = B_PER_W // LANES
ROWS_PER_VROW = 128 // FACTOR


def _fetch_rows(tab_ref, idx_v, r0, buf0, buf1, rows, sems):
    feat = lax.iota(jnp.int32, LANES)
    slots = [(buf0, 0), (buf0, 4), (buf1, 0), (buf1, 4)]

    def load16(g):
        row = r0 + g // 8
        col = (g % 8) * LANES
        return idx_v[row, pl.ds(col, LANES)]

    def fire4(rvec, q):
        buf, jb = slots[q]
        for jj in range(4):
            r = rvec[q * 4 + jj]
            tcb = pl.multiple_of((r // 128) * 128, 128)
            for t in range(4):
                pltpu.async_copy(
                    tab_ref.at[pl.ds(t * 8, 8), pl.ds(tcb, 128)],
                    buf.at[jb + jj].at[pl.ds(t * 8, 8)], sems[q])

    def wait4(q):
        buf, jb = slots[q]
        for jj in range(4):
            for t in range(4):
                pltpu.make_async_copy(
                    tab_ref.at[pl.ds(0, 8), pl.ds(0, 128)],
                    buf.at[jb + jj].at[pl.ds(t * 8, 8)], sems[q]).wait()

    def extract4(rvec, q, g):
        return
        buf, jb = slots[q]
        for jj in range(4):
            j = q * 4 + jj
            r = rvec[j]
            cv = jnp.full((LANES,), r % 128, dtype=jnp.int32)
            jv = jnp.full((LANES,), jb + jj, dtype=jnp.int32)
            lo = plsc.load_gather(buf, [jv, feat, cv])
            hi = plsc.load_gather(buf, [jv, feat + LANES, cv])
            kflat = g * LANES + j
            rw = kflat // ROWS_PER_VROW
            cb = (kflat % ROWS_PER_VROW) * FACTOR
            rows[rw, pl.ds(cb, LANES)] = lo
            rows[rw, pl.ds(cb + LANES, LANES)] = hi

    rvec0 = load16(0)
    for q in range(4):
        fire4(rvec0, q)

    def group_body(g, _):
        rvec = load16(g)
        rvnext = load16(jnp.minimum(g + 1, GROUPS - 1))
        for q in range(4):
            wait4(q)
            extract4(rvec, q, g)

            @pl.when(g < GROUPS - 1)
            def _():
                fire4(rvnext, q)

        return 0

    lax.fori_loop(0, GROUPS, group_body, 0)


def _mf_body(user_ref, item_ref, avg_ref, eu_ref, ei_ref,
             pred_ref, u_ref, it_ref,
             idx_u, idx_it, buf0, buf1, rows_u, rows_it, pred2, avg_v,
             sem0, sem1, sem2, sem3):
    wid = lax.axis_index("s") * NUM_CORES + lax.axis_index("c")
    crow8 = (wid // 2) * 8
    r0 = (wid % 2) * 4
    pltpu.sync_copy(user_ref.at[pl.ds(crow8, 8)], idx_u)
    pltpu.sync_copy(item_ref.at[pl.ds(crow8, 8)], idx_it)
    pltpu.sync_copy(avg_ref, avg_v)

    sems = (sem0, sem1, sem2, sem3)
    _fetch_rows(eu_ref, idx_u, r0, buf0, buf1, rows_u, sems)
    _fetch_rows(ei_ref, idx_it, r0, buf0, buf1, rows_it, sems)

    lane = lax.iota(jnp.int32, LANES)
    avg_vec = avg_v[...]

    def group_body(g, _):
        kvec = g * LANES + lane
        rvec = kvec // ROWS_PER_VROW
        cbase = (kvec % ROWS_PER_VROW) * FACTOR

        def col_body(c, acc):
            cc = cbase + c
            uv = plsc.load_gather(rows_u, [rvec, cc])
            iv = plsc.load_gather(rows_it, [rvec, cc])
            return acc + uv * iv

        acc = lax.fori_loop(0, FACTOR, col_body,
                            jnp.zeros((LANES,), jnp.float32))
        pred2[g // 2, pl.ds((g % 2) * LANES, LANES)] = acc + avg_vec
        return 0

    lax.fori_loop(0, GROUPS, group_body, 0)

    vbase = wid * (B_PER_W * FACTOR // 128)
    pltpu.sync_copy(rows_u, u_ref.at[pl.ds(vbase, 128)])
    pltpu.sync_copy(rows_it, it_ref.at[pl.ds(vbase, 128)])
    pltpu.sync_copy(pred2, pred_ref.at[pl.ds(wid * LANES, LANES)])


@jax.jit
def _mf(user2d, item2d, avg16, eu_t, ei_t):
    mesh = plsc.VectorSubcoreMesh(core_axis_name="c", subcore_axis_name="s")
    f32 = jnp.float32
    kern = pl.kernel(
        _mf_body,
        out_type=(
            jax.ShapeDtypeStruct((B_PER_W, FACTOR), f32),
            jax.ShapeDtypeStruct((BATCH * FACTOR // 128, 128), f32),
            jax.ShapeDtypeStruct((BATCH * FACTOR // 128, 128), f32),
        ),
        mesh=mesh,
        compiler_params=pltpu.CompilerParams(
            needs_layout_passes=False, use_tc_tiling_on_sc=True),
        scratch_types=[
            pltpu.VMEM((8, 128), jnp.int32),
            pltpu.VMEM((8, 128), jnp.int32),
            pltpu.VMEM((8, FACTOR, 128), f32),
            pltpu.VMEM((8, FACTOR, 128), f32),
            pltpu.VMEM((128, 128), f32),
            pltpu.VMEM((128, 128), f32),
            pltpu.VMEM((LANES, FACTOR), f32),
            pltpu.VMEM((LANES,), f32),
            pltpu.SemaphoreType.DMA,
            pltpu.SemaphoreType.DMA,
            pltpu.SemaphoreType.DMA,
            pltpu.SemaphoreType.DMA,
        ],
    )
    return kern(user2d, item2d, avg16, eu_t, ei_t)


def kernel(user, item, average, embed_user, embed_item):
    user2d = user.astype(jnp.int32).reshape(BATCH // 128, 128)
    item2d = item.astype(jnp.int32).reshape(BATCH // 128, 128)
    avg16 = jnp.broadcast_to(average.astype(jnp.float32), (LANES,))
    pred2, u4, it4 = _mf(user2d, item2d, avg16, embed_user.T, embed_item.T)
    return (pred2.reshape(BATCH),
            u4.reshape(BATCH, FACTOR),
            it4.reshape(BATCH, FACTOR))


# speedup vs baseline: 1.0002x; 1.0002x over previous
import functools

import jax
import jax.numpy as jnp
from jax import lax
from jax.experimental import pallas as pl
from jax.experimental.pallas import tpu as pltpu
from jax.experimental.pallas import tpu_sc as plsc

BATCH = 16384
FACTOR = 32
VOCAB = 1000000
NUM_CORES = 2
NUM_SUBCORES = 16
LANES = 16
NUM_WORKERS = NUM_CORES * NUM_SUBCORES
B_PER_W = BATCH // NUM_WORKERS
GROUPS = B_PER_W // LANES
ROWS_PER_VROW = 128 // FACTOR


def _fetch_rows(tab_ref, idx_v, r0, buf0, buf1, rows, sems):
    feat = lax.iota(jnp.int32, LANES)
    slots = [(buf0, 0), (buf0, 4), (buf1, 0), (buf1, 4)]

    def load16(g):
        row = r0 + g // 8
        col = (g % 8) * LANES
        return idx_v[row, pl.ds(col, LANES)]

    def fire4(rvec, q):
        buf, jb = slots[q]
        for jj in range(4):
            r = rvec[q * 4 + jj]
            tcb = pl.multiple_of((r // 128) * 128, 128)
            pltpu.async_copy(
                tab_ref.at[:, pl.ds(tcb, 128)], buf.at[jb + jj], sems[q])

    def wait4(q):
        buf, jb = slots[q]
        for jj in range(4):
            pltpu.make_async_copy(
                tab_ref.at[:, pl.ds(0, 128)], buf.at[jb + jj], sems[q]).wait()

    def extract4(rvec, q, g):
        buf, jb = slots[q]
        for jj in range(4):
            j = q * 4 + jj
            r = rvec[j]
            cv = jnp.full((LANES,), r % 128, dtype=jnp.int32)
            jv = jnp.full((LANES,), jb + jj, dtype=jnp.int32)
            lo = plsc.load_gather(buf, [jv, feat, cv])
            hi = plsc.load_gather(buf, [jv, feat + LANES, cv])
            kflat = g * LANES + j
            rw = kflat // ROWS_PER_VROW
            cb = (kflat % ROWS_PER_VROW) * FACTOR
            rows[rw, pl.ds(cb, LANES)] = lo
            rows[rw, pl.ds(cb + LANES, LANES)] = hi

    rvec0 = load16(0)
    for q in range(4):
        fire4(rvec0, q)

    def group_body(g, _):
        rvec = load16(g)
        rvnext = load16(jnp.minimum(g + 1, GROUPS - 1))
        for q in range(4):
            wait4(q)
            extract4(rvec, q, g)

            @pl.when(g < GROUPS - 1)
            def _():
                fire4(rvnext, q)

        return 0

    lax.fori_loop(0, GROUPS, group_body, 0)


def _mf_body(user_ref, item_ref, avg_ref, eu_ref, ei_ref,
             pred_ref, u_ref, it_ref,
             idx_u, idx_it, buf0, buf1, rows_u, rows_it, pred2, avg_v,
             sem0, sem1, sem2, sem3):
    wid = lax.axis_index("s") * NUM_CORES + lax.axis_index("c")
    crow8 = (wid // 2) * 8
    r0 = (wid % 2) * 4
    pltpu.sync_copy(user_ref.at[pl.ds(crow8, 8)], idx_u)
    pltpu.sync_copy(item_ref.at[pl.ds(crow8, 8)], idx_it)
    pltpu.sync_copy(avg_ref, avg_v)

    sems = (sem0, sem1, sem2, sem3)
    _fetch_rows(eu_ref, idx_u, r0, buf0, buf1, rows_u, sems)
    _fetch_rows(ei_ref, idx_it, r0, buf0, buf1, rows_it, sems)

    lane = lax.iota(jnp.int32, LANES)
    avg_vec = avg_v[...]

    def group_body(g, _):
        kvec = g * LANES + lane
        rvec = kvec // ROWS_PER_VROW
        cbase = (kvec % ROWS_PER_VROW) * FACTOR

        def col_body(c, acc):
            cc = cbase + c
            uv = plsc.load_gather(rows_u, [rvec, cc])
            iv = plsc.load_gather(rows_it, [rvec, cc])
            return acc + uv * iv

        acc = lax.fori_loop(0, FACTOR, col_body,
                            jnp.zeros((LANES,), jnp.float32))
        pred2[g // 2, pl.ds((g % 2) * LANES, LANES)] = acc + avg_vec
        return 0

    lax.fori_loop(0, GROUPS, group_body, 0)

    vbase = wid * (B_PER_W * FACTOR // 128)
    pltpu.sync_copy(rows_u, u_ref.at[pl.ds(vbase, 128)])
    pltpu.sync_copy(rows_it, it_ref.at[pl.ds(vbase, 128)])
    pltpu.sync_copy(pred2, pred_ref.at[pl.ds(wid * LANES, LANES)])


@jax.jit
def _mf(user2d, item2d, avg16, eu_t, ei_t):
    mesh = plsc.VectorSubcoreMesh(core_axis_name="c", subcore_axis_name="s")
    f32 = jnp.float32
    kern = pl.kernel(
        _mf_body,
        out_type=(
            jax.ShapeDtypeStruct((B_PER_W, FACTOR), f32),
            jax.ShapeDtypeStruct((BATCH * FACTOR // 128, 128), f32),
            jax.ShapeDtypeStruct((BATCH * FACTOR // 128, 128), f32),
        ),
        mesh=mesh,
        compiler_params=pltpu.CompilerParams(
            needs_layout_passes=False, use_tc_tiling_on_sc=True),
        scratch_types=[
            pltpu.VMEM((8, 128), jnp.int32),
            pltpu.VMEM((8, 128), jnp.int32),
            pltpu.VMEM((8, FACTOR, 128), f32),
            pltpu.VMEM((8, FACTOR, 128), f32),
            pltpu.VMEM((128, 128), f32),
            pltpu.VMEM((128, 128), f32),
            pltpu.VMEM((LANES, FACTOR), f32),
            pltpu.VMEM((LANES,), f32),
            pltpu.SemaphoreType.DMA,
            pltpu.SemaphoreType.DMA,
            pltpu.SemaphoreType.DMA,
            pltpu.SemaphoreType.DMA,
        ],
    )
    return kern(user2d, item2d, avg16, eu_t, ei_t)


def kernel(user, item, average, embed_user, embed_item):
    user2d = user.astype(jnp.int32).reshape(BATCH // 128, 128)
    item2d = item.astype(jnp.int32).reshape(BATCH // 128, 128)
    avg16 = jnp.broadcast_to(average.astype(jnp.float32), (LANES,))
    pred2, u4, it4 = _mf(user2d, item2d, avg16, embed_user.T, embed_item.T)
    return (pred2.reshape(BATCH),
            u4.reshape(BATCH, FACTOR),
            it4.reshape(BATCH, FACTOR))
